# baseline (speedup 1.0000x reference)
import functools

import jax
import jax.numpy as jnp
from jax import lax
from jax.experimental import pallas as pl
from jax.experimental.pallas import tpu as pltpu
from jax.experimental.pallas import tpu_sc as plsc

N = 10000
E = 320000
D_IN = 128
H = 5
C = 32
HC = 160
LP = 16

NC = 2
NS = 16
NW = NC * NS
PER_W = E // NW
CH = 80
NCH = PER_W // CH
NP = 10240
ROWS_T = NP // NS

_mesh = plsc.VectorSubcoreMesh(core_axis_name="c", subcore_axis_name="s")
_f32 = jnp.float32


def _splat(vec, lane):
    return vec.at[jnp.full((LP,), lane, jnp.int32)].get(mode="promise_in_bounds")


DR = 5
_JD = NCH // DR

_sc_params = pltpu.CompilerParams(use_tc_tiling_on_sc=False)


@functools.partial(
    pl.kernel,
    out_type=[
        jax.ShapeDtypeStruct((E, LP), _f32),
        jax.ShapeDtypeStruct((NC, NP, LP), _f32),
    ],
    mesh=_mesh,
    compiler_params=_sc_params,
    scratch_types=[
        pltpu.VMEM((PER_W,), _f32),
        pltpu.VMEM((LP,), _f32),
        pltpu.VMEM((DR, 1, CH), jnp.int32),
        pltpu.VMEM((DR, 1, CH), jnp.int32),
        pltpu.VMEM((DR, CH, LP), _f32),
        pltpu.VMEM((DR, CH, LP), _f32),
        pltpu.VMEM((DR, CH, LP), _f32),
        pltpu.VMEM_SHARED((NP, LP), _f32),
    ] + [pltpu.SemaphoreType.DMA] * (3 * DR + 1),
)
def _edge_logits(als, ald, src, dst, w, ce, zeros16, p_out, den_part,
                 wva, cev, srcc, dstc, arows, drows, pv, den_sh, *sems):
    semi = sems[:DR]
    semr = sems[DR:2 * DR]
    semw = sems[2 * DR:3 * DR]
    semsc = sems[3 * DR]
    cid = lax.axis_index("c")
    sid = lax.axis_index("s")
    wid = sid * NC + cid
    tbase = wid * PER_W
    pltpu.sync_copy(zeros16.at[pl.ds(sid * ROWS_T, ROWS_T)],
                    den_sh.at[pl.ds(sid * ROWS_T, ROWS_T)])
    pltpu.sync_copy(ce, cev)
    pltpu.sync_copy(w.at[pl.ds(tbase, PER_W)], wva)
    plsc.subcore_barrier()
    ce16 = cev[...]

    def _issue_idx(g, b):
        gb = tbase + g * CH
        pltpu.async_copy(src.at[pl.ds(gb, CH)], srcc.at[b, 0], semi[b])
        pltpu.async_copy(dst.at[pl.ds(gb, CH)], dstc.at[b, 0], semi[b])

    def _drain_idx(g, b):
        gb = tbase + g * CH
        pltpu.make_async_copy(src.at[pl.ds(gb, CH)], srcc.at[b, 0], semi[b]).wait()
        pltpu.make_async_copy(dst.at[pl.ds(gb, CH)], dstc.at[b, 0], semi[b]).wait()

    def _issue_rows(b):
        pltpu.async_copy(als.at[srcc.at[b, 0]], arows.at[b], semr[b])
        pltpu.async_copy(ald.at[dstc.at[b, 0]], drows.at[b], semr[b])

    def _drain_rows(b):
        pltpu.make_async_copy(als.at[srcc.at[b, 0]], arows.at[b], semr[b]).wait()
        pltpu.make_async_copy(ald.at[dstc.at[b, 0]], drows.at[b], semr[b]).wait()

    def _drain_writes(g, b):
        gb = tbase + g * CH
        pltpu.make_async_copy(pv.at[b], p_out.at[pl.ds(gb, CH)], semw[b]).wait()

    def _consume(g, b, last):
        _drain_rows(b)
        for k in range(CH // LP):
            w16 = wva[pl.ds(g * CH + k * LP, LP)]
            for l in range(LP):
                e = k * LP + l
                lg = arows[b, e, :] + drows[b, e, :] + _splat(w16, l) * ce16
                pv[b, e, :] = jnp.exp(jnp.where(lg >= 0, lg, 0.2 * lg))
        gb = tbase + g * CH
        pltpu.async_copy(pv.at[b], p_out.at[pl.ds(gb, CH)], semw[b])
        if last:
            pltpu.sync_copy(pv.at[b], den_sh.at[dstc.at[b, 0]], add=True)
            return None
        return pltpu.async_copy(pv.at[b], den_sh.at[dstc.at[b, 0]], semsc, add=True)

    for b in range(4):
        _issue_idx(b, b)
    _drain_idx(0, 0)
    _issue_rows(0)
    _drain_idx(1, 1)
    _issue_rows(1)

    def outer(j, carry):
        scat = None
        for b in range(DR):
            g = j * DR + b
            bi4 = (b + 4) % DR
            bi2 = (b + 2) % DR

            if scat is not None:
                scat.wait()

            @pl.when(jnp.logical_and(g >= 1, g + 4 < NCH))
            def _():
                _drain_writes(g - 1, bi4)

            @pl.when(g + 4 < NCH)
            def _():
                _issue_idx(g + 4, bi4)

            @pl.when(g + 2 < NCH)
            def _():
                _drain_idx(g + 2, bi2)
                _issue_rows(bi2)

            scat = _consume(g, b, last=(b == DR - 1))
        return carry

    lax.fori_loop(0, _JD, outer, 0)
    for b in range(DR):
        _drain_writes(NCH - DR + b, b)
    plsc.subcore_barrier()

    @pl.when(sid == 0)
    def _():
        pltpu.sync_copy(den_sh, den_part.at[cid])


CHB = 40
NCHB = PER_W // CHB
RB = 5
_JDB = NCHB // 10


def _make_aggr(store_alpha):
    outs = [jax.ShapeDtypeStruct((NC, NP, HC), _f32)]
    if store_alpha:
        outs = [jax.ShapeDtypeStruct((E, LP), _f32)] + outs

    @functools.partial(
        pl.kernel,
        out_type=outs,
        mesh=_mesh,
        compiler_params=_sc_params,
        scratch_types=[
            pltpu.VMEM((RB, 1, CHB), jnp.int32),
            pltpu.VMEM((RB, 1, CHB), jnp.int32),
            pltpu.VMEM((RB, CHB, LP), _f32),
            pltpu.VMEM((RB, CHB, LP), _f32),
            pltpu.VMEM((2, CHB, LP), _f32),
            pltpu.VMEM((2, CHB, HC), _f32),
            pltpu.VMEM_SHARED((NP, HC), _f32),
        ] + [pltpu.SemaphoreType.DMA] * (2 * RB + 5),
    )
    def _aggr(src, dst, p_in, rden, xs, zeros160, *rest):
        if store_alpha:
            alpha_out, out_part = rest[0], rest[1]
            rest = rest[2:]
        else:
            out_part = rest[0]
            rest = rest[1:]
        (srcc, dstc, prow, rrows, av, xrows) = rest[:6]
        out_sh = rest[6]
        sems = rest[7:]
        semi = sems[:RB]
        semr = sems[RB:2 * RB]
        semx = sems[2 * RB:2 * RB + 2]
        semwa = sems[2 * RB + 2:2 * RB + 4]
        semsc = sems[2 * RB + 4]
        cid = lax.axis_index("c")
        sid = lax.axis_index("s")
        wid = sid * NC + cid
        tbase = wid * PER_W
        pltpu.sync_copy(zeros160.at[pl.ds(sid * ROWS_T, ROWS_T)],
                        out_sh.at[pl.ds(sid * ROWS_T, ROWS_T)])
        plsc.subcore_barrier()

        def _issue_idx(g, b):
            gb = tbase + g * CHB
            pltpu.async_copy(src.at[pl.ds(gb, CHB)], srcc.at[b, 0], semi[b])
            pltpu.async_copy(dst.at[pl.ds(gb, CHB)], dstc.at[b, 0], semi[b])
            pltpu.async_copy(p_in.at[pl.ds(gb, CHB)], prow.at[b], semi[b])

        def _drain_idx(g, b):
            gb = tbase + g * CHB
            pltpu.make_async_copy(src.at[pl.ds(gb, CHB)], srcc.at[b, 0], semi[b]).wait()
            pltpu.make_async_copy(dst.at[pl.ds(gb, CHB)], dstc.at[b, 0], semi[b]).wait()
            pltpu.make_async_copy(p_in.at[pl.ds(gb, CHB)], prow.at[b], semi[b]).wait()

        def _issue_rden(b):
            pltpu.async_copy(rden.at[dstc.at[b, 0]], rrows.at[b], semr[b])

        def _drain_rden(b):
            pltpu.make_async_copy(rden.at[dstc.at[b, 0]], rrows.at[b], semr[b]).wait()

        def _issue_xs(b, par):
            pltpu.async_copy(xs.at[srcc.at[b, 0]], xrows.at[par], semx[par])

        def _drain_xs(b, par):
            pltpu.make_async_copy(xs.at[srcc.at[b, 0]], xrows.at[par], semx[par]).wait()

        def _drain_alpha(g, par):
            gb = tbase + g * CHB
            pltpu.make_async_copy(av.at[par], alpha_out.at[pl.ds(gb, CHB)], semwa[par]).wait()

        def _compute(g, b, par, last):
            def eo_body(eo, carry):
                for ei in range(8):
                    e = eo * 8 + ei
                    a = prow[b, e, :] * rrows[b, e, :]
                    if store_alpha:
                        av[par, e, :] = a
                    for h in range(H):
                        q = _splat(a, h)
                        for t in range(C // LP):
                            sl = pl.ds(h * C + t * LP, LP)
                            xrows[par, e, sl] = xrows[par, e, sl] * q
                return carry

            lax.fori_loop(0, CHB // 8, eo_body, 0)
            gb = tbase + g * CHB
            if store_alpha:
                pltpu.async_copy(av.at[par], alpha_out.at[pl.ds(gb, CHB)], semwa[par])
            if last:
                pltpu.sync_copy(xrows.at[par], out_sh.at[dstc.at[b, 0]], add=True)
                return None
            return pltpu.async_copy(xrows.at[par], out_sh.at[dstc.at[b, 0]],
                                    semsc, add=True)

        for b in range(4):
            _issue_idx(b, b)
        _drain_idx(0, 0)
        _issue_rden(0)
        _drain_idx(1, 1)
        _issue_rden(1)
        _issue_xs(0, 0)

        def outer(j, carry):
            scat = None
            for b in range(10):
                b5 = b % RB
                bm1 = (b - 1) % RB
                bi4 = (b + 4) % RB
                bi2 = (b + 2) % RB
                par = b % 2
                par1 = (b + 1) % 2
                g = j * 10 + b

                if scat is not None:
                    scat.wait()

                @pl.when(g + 4 < NCHB)
                def _():
                    _issue_idx(g + 4, bi4)

                @pl.when(g + 2 < NCHB)
                def _():
                    _drain_idx(g + 2, bi2)
                    _issue_rden(bi2)

                @pl.when(g + 1 < NCHB)
                def _():
                    _issue_xs((b + 1) % RB, par1)

                _drain_rden(b5)
                _drain_xs(b5, par)
                if store_alpha:
                    @pl.when(g >= 2)
                    def _():
                        _drain_alpha(g - 2, par)
                scat = _compute(g, b5, par, last=(b == 9))
            return carry

        lax.fori_loop(0, _JDB, outer, 0)
        if store_alpha:
            _drain_alpha(NCHB - 2, (NCHB - 2) % 2)
            _drain_alpha(NCHB - 1, (NCHB - 1) % 2)
        plsc.subcore_barrier()
        pltpu.sync_copy(out_sh.at[pl.ds(sid * ROWS_T, ROWS_T)],
                        out_part.at[cid, pl.ds(sid * ROWS_T, ROWS_T)])

    return _aggr


_aggr_plain = _make_aggr(False)
_aggr_alpha = _make_aggr(True)


_BN = 1000


def _proj_body(x_ref, w_ref, as_ref, bd_ref, proj_ref, asrc_ref, adst_ref):
    x = x_ref[...]
    t = jnp.dot(x, w_ref[...], preferred_element_type=_f32)
    proj_ref[...] = t
    asrc_ref[...] = jnp.dot(t, as_ref[...], preferred_element_type=_f32)
    adst_ref[...] = jnp.dot(x, bd_ref[...], preferred_element_type=_f32)


def _proj(x, w, amat_src, bmat_dst):
    k = x.shape[1]
    return pl.pallas_call(
        _proj_body,
        out_shape=[
            jax.ShapeDtypeStruct((N, HC), _f32),
            jax.ShapeDtypeStruct((N, LP), _f32),
            jax.ShapeDtypeStruct((N, LP), _f32),
        ],
        grid=(N // _BN,),
        in_specs=[
            pl.BlockSpec((_BN, k), lambda i: (i, 0)),
            pl.BlockSpec((k, HC), lambda i: (0, 0)),
            pl.BlockSpec((HC, LP), lambda i: (0, 0)),
            pl.BlockSpec((k, LP), lambda i: (0, 0)),
        ],
        out_specs=[
            pl.BlockSpec((_BN, HC), lambda i: (i, 0)),
            pl.BlockSpec((_BN, LP), lambda i: (i, 0)),
            pl.BlockSpec((_BN, LP), lambda i: (i, 0)),
        ],
    )(x, w, amat_src, bmat_dst)


def _rden_body(dp_ref, r_ref):
    r_ref[...] = 1.0 / (dp_ref[0] + dp_ref[1] + 1e-16)


def _rden(den_part):
    rb = 1024
    return pl.pallas_call(
        _rden_body,
        out_shape=jax.ShapeDtypeStruct((NP, LP), _f32),
        grid=(NP // rb,),
        in_specs=[pl.BlockSpec((NC, rb, LP), lambda i: (0, i, 0))],
        out_specs=pl.BlockSpec((rb, LP), lambda i: (i, 0)),
    )(den_part)


def _ln(x, g, b):
    mu = jnp.mean(x, axis=-1, keepdims=True)
    var = jnp.mean((x - mu) ** 2, axis=-1, keepdims=True)
    return (x - mu) * lax.rsqrt(var + 1e-5) * g + b


def _epi1_body(op_ref, b_ref, g_ref, be_ref, y_ref):
    o = op_ref[0] + op_ref[1] + b_ref[...]
    o = jnp.where(o >= 0, o, 0.01 * o)
    y_ref[...] = _ln(o, g_ref[...], be_ref[...])


def _epi1(out_part, b, g, be):
    return pl.pallas_call(
        _epi1_body,
        out_shape=jax.ShapeDtypeStruct((N, HC), _f32),
        grid=(N // _BN,),
        in_specs=[
            pl.BlockSpec((NC, _BN, HC), lambda i: (0, i, 0)),
            pl.BlockSpec((HC,), lambda i: (0,)),
            pl.BlockSpec((HC,), lambda i: (0,)),
            pl.BlockSpec((HC,), lambda i: (0,)),
        ],
        out_specs=pl.BlockSpec((_BN, HC), lambda i: (i, 0)),
    )(out_part, b, g, be)


def _epi2_body(op_ref, b_ref, g_ref, be_ref, y_ref):
    o = op_ref[0] + op_ref[1]
    m = o[:, 0:C]
    for h in range(1, H):
        m = m + o[:, h * C:(h + 1) * C]
    m = m * (1.0 / H) + b_ref[...]
    m = jnp.where(m >= 0, m, 0.01 * m)
    y_ref[...] = _ln(m, g_ref[...], be_ref[...])


def _epi2(out_part, b, g, be):
    return pl.pallas_call(
        _epi2_body,
        out_shape=jax.ShapeDtypeStruct((N, C), _f32),
        grid=(N // _BN,),
        in_specs=[
            pl.BlockSpec((NC, _BN, HC), lambda i: (0, i, 0)),
            pl.BlockSpec((C,), lambda i: (0,)),
            pl.BlockSpec((C,), lambda i: (0,)),
            pl.BlockSpec((C,), lambda i: (0,)),
        ],
        out_specs=pl.BlockSpec((_BN, C), lambda i: (i, 0)),
    )(out_part, b, g, be)


def _amat(a):
    e5 = jnp.eye(LP, dtype=_f32)[:H]
    return (a[0][:, :, None] * e5[:, None, :]).reshape(HC, LP)


def _ce_pad(le, ae):
    ce = jnp.sum(le.reshape(H, C) * ae[0], axis=-1)
    return jnp.concatenate([ce, jnp.zeros((LP - H,), _f32)])


def _direction(asrc, adst, s_idx, d_idx, w, cep, proj_src, zeros16, zeros160,
               store_alpha):
    p, den_part = _edge_logits(asrc, adst, s_idx, d_idx, w, cep, zeros16)
    rden = _rden(den_part)
    if store_alpha:
        alpha, out_part = _aggr_alpha(s_idx, d_idx, p, rden, proj_src, zeros160)
        return alpha, out_part
    out_part, = _aggr_plain(s_idx, d_idx, p, rden, proj_src, zeros160)
    return None, out_part


def kernel(x_s, x_t, edge_index, edge_weight, W1, le1, as1, ad1, ae1, b1, g1, be1, W2s, W2d, le2, as2, ad2, ae2, b2, g2, be2):
    src = edge_index[0].astype(jnp.int32)
    dst = edge_index[1].astype(jnp.int32)
    w = edge_weight[:, 0]
    zeros16 = jnp.zeros((NP, LP), _f32)
    zeros160 = jnp.zeros((NP, HC), _f32)

    As1 = _amat(as1)
    Ad1 = _amat(ad1)
    cep1 = _ce_pad(le1, ae1)
    proj_s, asrc_s, adst_s = _proj(x_s, W1, As1, W1 @ Ad1)
    proj_t, asrc_t, adst_t = _proj(x_t, W1, As1, W1 @ Ad1)

    _, op_t = _direction(asrc_s, adst_t, src, dst, w, cep1, proj_s,
                         zeros16, zeros160, False)
    _, op_s = _direction(asrc_t, adst_s, dst, src, w, cep1, proj_t,
                         zeros16, zeros160, False)
    x_t1 = _epi1(op_t, b1, g1, be1)
    x_s1 = _epi1(op_s, b1, g1, be1)

    As2 = _amat(as2)
    Ad2 = _amat(ad2)
    cep2 = _ce_pad(le2, ae2)
    Bd2 = W2d @ Ad2
    proj2_s, asrc2_s, adst2_s = _proj(x_s1, W2s, As2, Bd2)
    proj2_t, asrc2_t, adst2_t = _proj(x_t1, W2s, As2, Bd2)

    al_t, op_t2 = _direction(asrc2_s, adst2_t, src, dst, w, cep2, proj2_s,
                             zeros16, zeros160, True)
    al_s, op_s2 = _direction(asrc2_t, adst2_s, dst, src, w, cep2, proj2_t,
                             zeros16, zeros160, True)
    x_t2 = _epi2(op_t2, b2, g2, be2)
    x_s2 = _epi2(op_s2, b2, g2, be2)
    at_t = al_t[:, :H]
    at_s = al_s[:, :H]
    return (x_s2, at_s, x_t2, at_t)


# speedup vs baseline: 1.7672x; 1.7672x over previous
import functools

import jax
import jax.numpy as jnp
from jax import lax
from jax.experimental import pallas as pl
from jax.experimental.pallas import tpu as pltpu
from jax.experimental.pallas import tpu_sc as plsc

N = 10000
E = 320000
D_IN = 128
H = 5
C = 32
HC = 160
LP = 16

NC = 2
NS = 16
NW = NC * NS
PER_W = E // NW
CH = 80
NCH = PER_W // CH
NP = 10240
ROWS_T = NP // NS

_mesh = plsc.VectorSubcoreMesh(core_axis_name="c", subcore_axis_name="s")
_f32 = jnp.float32


def _splat(vec, lane):
    return vec.at[jnp.full((LP,), lane, jnp.int32)].get(mode="promise_in_bounds")


DR = 5
_JD = NCH // DR

_sc_params = pltpu.CompilerParams(use_tc_tiling_on_sc=False)


@functools.partial(
    pl.kernel,
    out_type=[
        jax.ShapeDtypeStruct((E, LP), _f32),
        jax.ShapeDtypeStruct((NC, NP, LP), _f32),
    ],
    mesh=_mesh,
    compiler_params=_sc_params,
    scratch_types=[
        pltpu.VMEM((PER_W,), _f32),
        pltpu.VMEM((LP,), _f32),
        pltpu.VMEM((DR, 1, CH), jnp.int32),
        pltpu.VMEM((DR, 1, CH), jnp.int32),
        pltpu.VMEM((DR, CH, LP), _f32),
        pltpu.VMEM((DR, CH, LP), _f32),
        pltpu.VMEM((DR, CH, LP), _f32),
        pltpu.VMEM_SHARED((NP, LP), _f32),
    ] + [pltpu.SemaphoreType.DMA] * (3 * DR + 1),
)
def _edge_logits(als, ald, src, dst, w, ce, zeros16, p_out, den_part,
                 wva, cev, srcc, dstc, arows, drows, pv, den_sh, *sems):
    semi = sems[:DR]
    semr = sems[DR:2 * DR]
    semw = sems[2 * DR:3 * DR]
    semsc = sems[3 * DR]
    cid = lax.axis_index("c")
    sid = lax.axis_index("s")
    wid = sid * NC + cid
    tbase = wid * PER_W
    pltpu.sync_copy(zeros16.at[pl.ds(sid * ROWS_T, ROWS_T)],
                    den_sh.at[pl.ds(sid * ROWS_T, ROWS_T)])
    pltpu.sync_copy(ce, cev)
    pltpu.sync_copy(w.at[pl.ds(tbase, PER_W)], wva)
    plsc.subcore_barrier()
    ce16 = cev[...]

    def _issue_idx(g, b):
        gb = tbase + g * CH
        pltpu.async_copy(src.at[pl.ds(gb, CH)], srcc.at[b, 0], semi[b])
        pltpu.async_copy(dst.at[pl.ds(gb, CH)], dstc.at[b, 0], semi[b])

    def _drain_idx(g, b):
        gb = tbase + g * CH
        pltpu.make_async_copy(src.at[pl.ds(gb, CH)], srcc.at[b, 0], semi[b]).wait()
        pltpu.make_async_copy(dst.at[pl.ds(gb, CH)], dstc.at[b, 0], semi[b]).wait()

    def _issue_rows(b):
        pltpu.async_copy(als.at[srcc.at[b, 0]], arows.at[b], semr[b])
        pltpu.async_copy(ald.at[dstc.at[b, 0]], drows.at[b], semr[b])

    def _drain_rows(b):
        pltpu.make_async_copy(als.at[srcc.at[b, 0]], arows.at[b], semr[b]).wait()
        pltpu.make_async_copy(ald.at[dstc.at[b, 0]], drows.at[b], semr[b]).wait()

    def _drain_writes(g, b):
        gb = tbase + g * CH
        pltpu.make_async_copy(pv.at[b], p_out.at[pl.ds(gb, CH)], semw[b]).wait()

    def _consume(g, b, last):
        _drain_rows(b)
        for k in range(CH // LP):
            w16 = wva[pl.ds(g * CH + k * LP, LP)]
            for l in range(LP):
                e = k * LP + l
                lg = arows[b, e, :] + drows[b, e, :] + _splat(w16, l) * ce16
                pv[b, e, :] = jnp.exp(jnp.where(lg >= 0, lg, 0.2 * lg))
        gb = tbase + g * CH
        pltpu.async_copy(pv.at[b], p_out.at[pl.ds(gb, CH)], semw[b])
        if last:
            pltpu.sync_copy(pv.at[b], den_sh.at[dstc.at[b, 0]], add=True)
            return None
        return pltpu.async_copy(pv.at[b], den_sh.at[dstc.at[b, 0]], semsc, add=True)

    for b in range(4):
        _issue_idx(b, b)
    _drain_idx(0, 0)
    _issue_rows(0)
    _drain_idx(1, 1)
    _issue_rows(1)

    def outer(j, carry):
        scat = None
        for b in range(DR):
            g = j * DR + b
            bi4 = (b + 4) % DR
            bi2 = (b + 2) % DR

            if scat is not None:
                scat.wait()

            @pl.when(jnp.logical_and(g >= 1, g + 4 < NCH))
            def _():
                _drain_writes(g - 1, bi4)

            @pl.when(g + 4 < NCH)
            def _():
                _issue_idx(g + 4, bi4)

            @pl.when(g + 2 < NCH)
            def _():
                _drain_idx(g + 2, bi2)
                _issue_rows(bi2)

            scat = _consume(g, b, last=(b == DR - 1))
        return carry

    lax.fori_loop(0, _JD, outer, 0)
    for b in range(DR):
        _drain_writes(NCH - DR + b, b)
    plsc.subcore_barrier()

    @pl.when(sid == 0)
    def _():
        pltpu.sync_copy(den_sh, den_part.at[cid])


CHB = 40
NCHB = PER_W // CHB
RB = 5
_JDB = NCHB // 10


def _make_aggr(store_alpha):
    outs = [jax.ShapeDtypeStruct((NC, NP, HC), _f32)]
    if store_alpha:
        outs = [jax.ShapeDtypeStruct((E, LP), _f32)] + outs

    @functools.partial(
        pl.kernel,
        out_type=outs,
        mesh=_mesh,
        compiler_params=_sc_params,
        scratch_types=[
            pltpu.VMEM((RB, 1, CHB), jnp.int32),
            pltpu.VMEM((RB, 1, CHB), jnp.int32),
            pltpu.VMEM((RB, CHB, LP), _f32),
            pltpu.VMEM((RB, CHB, LP), _f32),
            pltpu.VMEM((2, CHB, LP), _f32),
            pltpu.VMEM((2, CHB, HC), _f32),
            pltpu.VMEM_SHARED((NP, HC), _f32),
        ] + [pltpu.SemaphoreType.DMA] * (2 * RB + 5),
    )
    def _aggr(src, dst, p_in, rden, xs, zeros160, *rest):
        if store_alpha:
            alpha_out, out_part = rest[0], rest[1]
            rest = rest[2:]
        else:
            out_part = rest[0]
            rest = rest[1:]
        (srcc, dstc, prow, rrows, av, xrows) = rest[:6]
        out_sh = rest[6]
        sems = rest[7:]
        semi = sems[:RB]
        semr = sems[RB:2 * RB]
        semx = sems[2 * RB:2 * RB + 2]
        semwa = sems[2 * RB + 2:2 * RB + 4]
        semsc = sems[2 * RB + 4]
        cid = lax.axis_index("c")
        sid = lax.axis_index("s")
        wid = sid * NC + cid
        tbase = wid * PER_W
        pltpu.sync_copy(zeros160.at[pl.ds(sid * ROWS_T, ROWS_T)],
                        out_sh.at[pl.ds(sid * ROWS_T, ROWS_T)])
        plsc.subcore_barrier()

        def _issue_idx(g, b):
            gb = tbase + g * CHB
            pltpu.async_copy(src.at[pl.ds(gb, CHB)], srcc.at[b, 0], semi[b])
            pltpu.async_copy(dst.at[pl.ds(gb, CHB)], dstc.at[b, 0], semi[b])
            pltpu.async_copy(p_in.at[pl.ds(gb, CHB)], prow.at[b], semi[b])

        def _drain_idx(g, b):
            gb = tbase + g * CHB
            pltpu.make_async_copy(src.at[pl.ds(gb, CHB)], srcc.at[b, 0], semi[b]).wait()
            pltpu.make_async_copy(dst.at[pl.ds(gb, CHB)], dstc.at[b, 0], semi[b]).wait()
            pltpu.make_async_copy(p_in.at[pl.ds(gb, CHB)], prow.at[b], semi[b]).wait()

        def _issue_rden(b):
            pltpu.async_copy(rden.at[dstc.at[b, 0]], rrows.at[b], semr[b])

        def _drain_rden(b):
            pltpu.make_async_copy(rden.at[dstc.at[b, 0]], rrows.at[b], semr[b]).wait()

        def _issue_xs(b, par):
            pltpu.async_copy(xs.at[srcc.at[b, 0]], xrows.at[par], semx[par])

        def _drain_xs(b, par):
            pltpu.make_async_copy(xs.at[srcc.at[b, 0]], xrows.at[par], semx[par]).wait()

        def _drain_alpha(g, par):
            gb = tbase + g * CHB
            pltpu.make_async_copy(av.at[par], alpha_out.at[pl.ds(gb, CHB)], semwa[par]).wait()

        def _compute(g, b, par, last):
            @plsc.parallel_loop(0, CHB, step=1, unroll=8)
            def _edges(e):
                a = prow[b, e, :] * rrows[b, e, :]
                if store_alpha:
                    av[par, e, :] = a
                for h in range(H):
                    q = _splat(a, h)
                    for t in range(C // LP):
                        sl = pl.ds(h * C + t * LP, LP)
                        xrows[par, e, sl] = xrows[par, e, sl] * q
            gb = tbase + g * CHB
            if store_alpha:
                pltpu.async_copy(av.at[par], alpha_out.at[pl.ds(gb, CHB)], semwa[par])
            if last:
                pltpu.sync_copy(xrows.at[par], out_sh.at[dstc.at[b, 0]], add=True)
                return None
            return pltpu.async_copy(xrows.at[par], out_sh.at[dstc.at[b, 0]],
                                    semsc, add=True)

        for b in range(4):
            _issue_idx(b, b)
        _drain_idx(0, 0)
        _issue_rden(0)
        _drain_idx(1, 1)
        _issue_rden(1)
        _issue_xs(0, 0)

        def outer(j, carry):
            scat = None
            for b in range(10):
                b5 = b % RB
                bm1 = (b - 1) % RB
                bi4 = (b + 4) % RB
                bi2 = (b + 2) % RB
                par = b % 2
                par1 = (b + 1) % 2
                g = j * 10 + b

                if scat is not None:
                    scat.wait()

                @pl.when(g + 4 < NCHB)
                def _():
                    _issue_idx(g + 4, bi4)

                @pl.when(g + 2 < NCHB)
                def _():
                    _drain_idx(g + 2, bi2)
                    _issue_rden(bi2)

                @pl.when(g + 1 < NCHB)
                def _():
                    _issue_xs((b + 1) % RB, par1)

                _drain_rden(b5)
                _drain_xs(b5, par)
                if store_alpha:
                    @pl.when(g >= 2)
                    def _():
                        _drain_alpha(g - 2, par)
                scat = _compute(g, b5, par, last=(b == 9))
            return carry

        lax.fori_loop(0, _JDB, outer, 0)
        if store_alpha:
            _drain_alpha(NCHB - 2, (NCHB - 2) % 2)
            _drain_alpha(NCHB - 1, (NCHB - 1) % 2)
        plsc.subcore_barrier()
        pltpu.sync_copy(out_sh.at[pl.ds(sid * ROWS_T, ROWS_T)],
                        out_part.at[cid, pl.ds(sid * ROWS_T, ROWS_T)])

    return _aggr


_aggr_plain = _make_aggr(False)
_aggr_alpha = _make_aggr(True)


_BN = 1000


def _proj_body(x_ref, w_ref, as_ref, bd_ref, proj_ref, asrc_ref, adst_ref):
    x = x_ref[...]
    t = jnp.dot(x, w_ref[...], preferred_element_type=_f32)
    proj_ref[...] = t
    asrc_ref[...] = jnp.dot(t, as_ref[...], preferred_element_type=_f32)
    adst_ref[...] = jnp.dot(x, bd_ref[...], preferred_element_type=_f32)


def _proj(x, w, amat_src, bmat_dst):
    k = x.shape[1]
    return pl.pallas_call(
        _proj_body,
        out_shape=[
            jax.ShapeDtypeStruct((N, HC), _f32),
            jax.ShapeDtypeStruct((N, LP), _f32),
            jax.ShapeDtypeStruct((N, LP), _f32),
        ],
        grid=(N // _BN,),
        in_specs=[
            pl.BlockSpec((_BN, k), lambda i: (i, 0)),
            pl.BlockSpec((k, HC), lambda i: (0, 0)),
            pl.BlockSpec((HC, LP), lambda i: (0, 0)),
            pl.BlockSpec((k, LP), lambda i: (0, 0)),
        ],
        out_specs=[
            pl.BlockSpec((_BN, HC), lambda i: (i, 0)),
            pl.BlockSpec((_BN, LP), lambda i: (i, 0)),
            pl.BlockSpec((_BN, LP), lambda i: (i, 0)),
        ],
    )(x, w, amat_src, bmat_dst)


def _rden_body(dp_ref, r_ref):
    r_ref[...] = 1.0 / (dp_ref[0] + dp_ref[1] + 1e-16)


def _rden(den_part):
    rb = 1024
    return pl.pallas_call(
        _rden_body,
        out_shape=jax.ShapeDtypeStruct((NP, LP), _f32),
        grid=(NP // rb,),
        in_specs=[pl.BlockSpec((NC, rb, LP), lambda i: (0, i, 0))],
        out_specs=pl.BlockSpec((rb, LP), lambda i: (i, 0)),
    )(den_part)


def _ln(x, g, b):
    mu = jnp.mean(x, axis=-1, keepdims=True)
    var = jnp.mean((x - mu) ** 2, axis=-1, keepdims=True)
    return (x - mu) * lax.rsqrt(var + 1e-5) * g + b


def _epi1_body(op_ref, b_ref, g_ref, be_ref, y_ref):
    o = op_ref[0] + op_ref[1] + b_ref[...]
    o = jnp.where(o >= 0, o, 0.01 * o)
    y_ref[...] = _ln(o, g_ref[...], be_ref[...])


def _epi1(out_part, b, g, be):
    return pl.pallas_call(
        _epi1_body,
        out_shape=jax.ShapeDtypeStruct((N, HC), _f32),
        grid=(N // _BN,),
        in_specs=[
            pl.BlockSpec((NC, _BN, HC), lambda i: (0, i, 0)),
            pl.BlockSpec((HC,), lambda i: (0,)),
            pl.BlockSpec((HC,), lambda i: (0,)),
            pl.BlockSpec((HC,), lambda i: (0,)),
        ],
        out_specs=pl.BlockSpec((_BN, HC), lambda i: (i, 0)),
    )(out_part, b, g, be)


def _epi2_body(op_ref, b_ref, g_ref, be_ref, y_ref):
    o = op_ref[0] + op_ref[1]
    m = o[:, 0:C]
    for h in range(1, H):
        m = m + o[:, h * C:(h + 1) * C]
    m = m * (1.0 / H) + b_ref[...]
    m = jnp.where(m >= 0, m, 0.01 * m)
    y_ref[...] = _ln(m, g_ref[...], be_ref[...])


def _epi2(out_part, b, g, be):
    return pl.pallas_call(
        _epi2_body,
        out_shape=jax.ShapeDtypeStruct((N, C), _f32),
        grid=(N // _BN,),
        in_specs=[
            pl.BlockSpec((NC, _BN, HC), lambda i: (0, i, 0)),
            pl.BlockSpec((C,), lambda i: (0,)),
            pl.BlockSpec((C,), lambda i: (0,)),
            pl.BlockSpec((C,), lambda i: (0,)),
        ],
        out_specs=pl.BlockSpec((_BN, C), lambda i: (i, 0)),
    )(out_part, b, g, be)


def _amat(a):
    e5 = jnp.eye(LP, dtype=_f32)[:H]
    return (a[0][:, :, None] * e5[:, None, :]).reshape(HC, LP)


def _ce_pad(le, ae):
    ce = jnp.sum(le.reshape(H, C) * ae[0], axis=-1)
    return jnp.concatenate([ce, jnp.zeros((LP - H,), _f32)])


def _direction(asrc, adst, s_idx, d_idx, w, cep, proj_src, zeros16, zeros160,
               store_alpha):
    p, den_part = _edge_logits(asrc, adst, s_idx, d_idx, w, cep, zeros16)
    rden = _rden(den_part)
    if store_alpha:
        alpha, out_part = _aggr_alpha(s_idx, d_idx, p, rden, proj_src, zeros160)
        return alpha, out_part
    out_part, = _aggr_plain(s_idx, d_idx, p, rden, proj_src, zeros160)
    return None, out_part


def kernel(x_s, x_t, edge_index, edge_weight, W1, le1, as1, ad1, ae1, b1, g1, be1, W2s, W2d, le2, as2, ad2, ae2, b2, g2, be2):
    src = edge_index[0].astype(jnp.int32)
    dst = edge_index[1].astype(jnp.int32)
    w = edge_weight[:, 0]
    zeros16 = jnp.zeros((NP, LP), _f32)
    zeros160 = jnp.zeros((NP, HC), _f32)

    As1 = _amat(as1)
    Ad1 = _amat(ad1)
    cep1 = _ce_pad(le1, ae1)
    proj_s, asrc_s, adst_s = _proj(x_s, W1, As1, W1 @ Ad1)
    proj_t, asrc_t, adst_t = _proj(x_t, W1, As1, W1 @ Ad1)

    _, op_t = _direction(asrc_s, adst_t, src, dst, w, cep1, proj_s,
                         zeros16, zeros160, False)
    _, op_s = _direction(asrc_t, adst_s, dst, src, w, cep1, proj_t,
                         zeros16, zeros160, False)
    x_t1 = _epi1(op_t, b1, g1, be1)
    x_s1 = _epi1(op_s, b1, g1, be1)

    As2 = _amat(as2)
    Ad2 = _amat(ad2)
    cep2 = _ce_pad(le2, ae2)
    Bd2 = W2d @ Ad2
    proj2_s, asrc2_s, adst2_s = _proj(x_s1, W2s, As2, Bd2)
    proj2_t, asrc2_t, adst2_t = _proj(x_t1, W2s, As2, Bd2)

    al_t, op_t2 = _direction(asrc2_s, adst2_t, src, dst, w, cep2, proj2_s,
                             zeros16, zeros160, True)
    al_s, op_s2 = _direction(asrc2_t, adst2_s, dst, src, w, cep2, proj2_t,
                             zeros16, zeros160, True)
    x_t2 = _epi2(op_t2, b2, g2, be2)
    x_s2 = _epi2(op_s2, b2, g2, be2)
    at_t = al_t[:, :H]
    at_s = al_s[:, :H]
    return (x_s2, at_s, x_t2, at_t)


# speedup vs baseline: 1.8049x; 1.0213x over previous
import functools

import jax
import jax.numpy as jnp
from jax import lax
from jax.experimental import pallas as pl
from jax.experimental.pallas import tpu as pltpu
from jax.experimental.pallas import tpu_sc as plsc

N = 10000
E = 320000
D_IN = 128
H = 5
C = 32
HC = 160
LP = 16

NC = 2
NS = 16
NW = NC * NS
PER_W = E // NW
CH = 80
NCH = PER_W // CH
NP = 10240
ROWS_T = NP // NS

_mesh = plsc.VectorSubcoreMesh(core_axis_name="c", subcore_axis_name="s")
_f32 = jnp.float32


def _splat(vec, lane):
    return vec.at[jnp.full((LP,), lane, jnp.int32)].get(mode="promise_in_bounds")


DR = 5
_JD = NCH // DR

_sc_params = pltpu.CompilerParams(use_tc_tiling_on_sc=False)


@functools.partial(
    pl.kernel,
    out_type=[
        jax.ShapeDtypeStruct((E, LP), _f32),
        jax.ShapeDtypeStruct((NC, NP, LP), _f32),
    ],
    mesh=_mesh,
    compiler_params=_sc_params,
    scratch_types=[
        pltpu.VMEM((PER_W,), _f32),
        pltpu.VMEM((LP,), _f32),
        pltpu.VMEM((DR, 1, CH), jnp.int32),
        pltpu.VMEM((DR, 1, CH), jnp.int32),
        pltpu.VMEM((DR, CH, LP), _f32),
        pltpu.VMEM((DR, CH, LP), _f32),
        pltpu.VMEM((DR, CH, LP), _f32),
        pltpu.VMEM_SHARED((NP, LP), _f32),
    ] + [pltpu.SemaphoreType.DMA] * (3 * DR + 1),
)
def _edge_logits(als, ald, src, dst, w, ce, zeros16, p_out, den_part,
                 wva, cev, srcc, dstc, arows, drows, pv, den_sh, *sems):
    semi = sems[:DR]
    semr = sems[DR:2 * DR]
    semw = sems[2 * DR:3 * DR]
    semsc = sems[3 * DR]
    cid = lax.axis_index("c")
    sid = lax.axis_index("s")
    wid = sid * NC + cid
    tbase = wid * PER_W
    pltpu.sync_copy(zeros16.at[pl.ds(sid * ROWS_T, ROWS_T)],
                    den_sh.at[pl.ds(sid * ROWS_T, ROWS_T)])
    pltpu.sync_copy(ce, cev)
    pltpu.sync_copy(w.at[pl.ds(tbase, PER_W)], wva)
    plsc.subcore_barrier()
    ce16 = cev[...]

    def _issue_idx(g, b):
        gb = tbase + g * CH
        pltpu.async_copy(src.at[pl.ds(gb, CH)], srcc.at[b, 0], semi[b])
        pltpu.async_copy(dst.at[pl.ds(gb, CH)], dstc.at[b, 0], semi[b])

    def _drain_idx(g, b):
        gb = tbase + g * CH
        pltpu.make_async_copy(src.at[pl.ds(gb, CH)], srcc.at[b, 0], semi[b]).wait()
        pltpu.make_async_copy(dst.at[pl.ds(gb, CH)], dstc.at[b, 0], semi[b]).wait()

    def _issue_rows(b):
        pltpu.async_copy(als.at[srcc.at[b, 0]], arows.at[b], semr[b])
        pltpu.async_copy(ald.at[dstc.at[b, 0]], drows.at[b], semr[b])

    def _drain_rows(b):
        pltpu.make_async_copy(als.at[srcc.at[b, 0]], arows.at[b], semr[b]).wait()
        pltpu.make_async_copy(ald.at[dstc.at[b, 0]], drows.at[b], semr[b]).wait()

    def _drain_writes(g, b):
        gb = tbase + g * CH
        pltpu.make_async_copy(pv.at[b], p_out.at[pl.ds(gb, CH)], semw[b]).wait()

    def _consume(g, b, last):
        _drain_rows(b)

        @plsc.parallel_loop(0, CH // LP, step=1, unroll=5)
        def _groups(k):
            w16 = wva[pl.ds(g * CH + k * LP, LP)]
            for l in range(LP):
                e = k * LP + l
                lg = arows[b, e, :] + drows[b, e, :] + _splat(w16, l) * ce16
                pv[b, e, :] = jnp.exp(jnp.where(lg >= 0, lg, 0.2 * lg))
        gb = tbase + g * CH
        pltpu.async_copy(pv.at[b], p_out.at[pl.ds(gb, CH)], semw[b])
        if last:
            pltpu.sync_copy(pv.at[b], den_sh.at[dstc.at[b, 0]], add=True)
            return None
        return pltpu.async_copy(pv.at[b], den_sh.at[dstc.at[b, 0]], semsc, add=True)

    for b in range(4):
        _issue_idx(b, b)
    _drain_idx(0, 0)
    _issue_rows(0)
    _drain_idx(1, 1)
    _issue_rows(1)

    def outer(j, carry):
        scat = None
        for b in range(DR):
            g = j * DR + b
            bi4 = (b + 4) % DR
            bi2 = (b + 2) % DR

            if scat is not None:
                scat.wait()

            @pl.when(jnp.logical_and(g >= 1, g + 4 < NCH))
            def _():
                _drain_writes(g - 1, bi4)

            @pl.when(g + 4 < NCH)
            def _():
                _issue_idx(g + 4, bi4)

            @pl.when(g + 2 < NCH)
            def _():
                _drain_idx(g + 2, bi2)
                _issue_rows(bi2)

            scat = _consume(g, b, last=(b == DR - 1))
        return carry

    lax.fori_loop(0, _JD, outer, 0)
    for b in range(DR):
        _drain_writes(NCH - DR + b, b)
    plsc.subcore_barrier()

    @pl.when(sid == 0)
    def _():
        pltpu.sync_copy(den_sh, den_part.at[cid])


CHB = 40
NCHB = PER_W // CHB
RB = 5
_JDB = NCHB // 10


def _make_aggr(store_alpha):
    outs = [jax.ShapeDtypeStruct((NC, NP, HC), _f32)]
    if store_alpha:
        outs = [jax.ShapeDtypeStruct((E, LP), _f32)] + outs

    @functools.partial(
        pl.kernel,
        out_type=outs,
        mesh=_mesh,
        compiler_params=_sc_params,
        scratch_types=[
            pltpu.VMEM((RB, 1, CHB), jnp.int32),
            pltpu.VMEM((RB, 1, CHB), jnp.int32),
            pltpu.VMEM((RB, CHB, LP), _f32),
            pltpu.VMEM((RB, CHB, LP), _f32),
            pltpu.VMEM((2, CHB, LP), _f32),
            pltpu.VMEM((2, CHB, HC), _f32),
            pltpu.VMEM_SHARED((NP, HC), _f32),
        ] + [pltpu.SemaphoreType.DMA] * (2 * RB + 5),
    )
    def _aggr(src, dst, p_in, rden, xs, zeros160, *rest):
        if store_alpha:
            alpha_out, out_part = rest[0], rest[1]
            rest = rest[2:]
        else:
            out_part = rest[0]
            rest = rest[1:]
        (srcc, dstc, prow, rrows, av, xrows) = rest[:6]
        out_sh = rest[6]
        sems = rest[7:]
        semi = sems[:RB]
        semr = sems[RB:2 * RB]
        semx = sems[2 * RB:2 * RB + 2]
        semwa = sems[2 * RB + 2:2 * RB + 4]
        semsc = sems[2 * RB + 4]
        cid = lax.axis_index("c")
        sid = lax.axis_index("s")
        wid = sid * NC + cid
        tbase = wid * PER_W
        pltpu.sync_copy(zeros160.at[pl.ds(sid * ROWS_T, ROWS_T)],
                        out_sh.at[pl.ds(sid * ROWS_T, ROWS_T)])
        plsc.subcore_barrier()

        def _issue_idx(g, b):
            gb = tbase + g * CHB
            pltpu.async_copy(src.at[pl.ds(gb, CHB)], srcc.at[b, 0], semi[b])
            pltpu.async_copy(dst.at[pl.ds(gb, CHB)], dstc.at[b, 0], semi[b])
            pltpu.async_copy(p_in.at[pl.ds(gb, CHB)], prow.at[b], semi[b])

        def _drain_idx(g, b):
            gb = tbase + g * CHB
            pltpu.make_async_copy(src.at[pl.ds(gb, CHB)], srcc.at[b, 0], semi[b]).wait()
            pltpu.make_async_copy(dst.at[pl.ds(gb, CHB)], dstc.at[b, 0], semi[b]).wait()
            pltpu.make_async_copy(p_in.at[pl.ds(gb, CHB)], prow.at[b], semi[b]).wait()

        def _issue_rden(b):
            pltpu.async_copy(rden.at[dstc.at[b, 0]], rrows.at[b], semr[b])

        def _drain_rden(b):
            pltpu.make_async_copy(rden.at[dstc.at[b, 0]], rrows.at[b], semr[b]).wait()

        def _issue_xs(b, par):
            pltpu.async_copy(xs.at[srcc.at[b, 0]], xrows.at[par], semx[par])

        def _drain_xs(b, par):
            pltpu.make_async_copy(xs.at[srcc.at[b, 0]], xrows.at[par], semx[par]).wait()

        def _drain_alpha(g, par):
            gb = tbase + g * CHB
            pltpu.make_async_copy(av.at[par], alpha_out.at[pl.ds(gb, CHB)], semwa[par]).wait()

        def _compute(g, b, par, last):
            @plsc.parallel_loop(0, CHB, step=1, unroll=8)
            def _edges(e):
                a = prow[b, e, :] * rrows[b, e, :]
                if store_alpha:
                    av[par, e, :] = a
                for h in range(H):
                    q = _splat(a, h)
                    for t in range(C // LP):
                        sl = pl.ds(h * C + t * LP, LP)
                        xrows[par, e, sl] = xrows[par, e, sl] * q
            gb = tbase + g * CHB
            if store_alpha:
                pltpu.async_copy(av.at[par], alpha_out.at[pl.ds(gb, CHB)], semwa[par])
            if last:
                pltpu.sync_copy(xrows.at[par], out_sh.at[dstc.at[b, 0]], add=True)
                return None
            return pltpu.async_copy(xrows.at[par], out_sh.at[dstc.at[b, 0]],
                                    semsc, add=True)

        for b in range(4):
            _issue_idx(b, b)
        _drain_idx(0, 0)
        _issue_rden(0)
        _drain_idx(1, 1)
        _issue_rden(1)
        _issue_xs(0, 0)

        def outer(j, carry):
            scat = None
            for b in range(10):
                b5 = b % RB
                bm1 = (b - 1) % RB
                bi4 = (b + 4) % RB
                bi2 = (b + 2) % RB
                par = b % 2
                par1 = (b + 1) % 2
                g = j * 10 + b

                if scat is not None:
                    scat.wait()

                @pl.when(g + 4 < NCHB)
                def _():
                    _issue_idx(g + 4, bi4)

                @pl.when(g + 2 < NCHB)
                def _():
                    _drain_idx(g + 2, bi2)
                    _issue_rden(bi2)

                @pl.when(g + 1 < NCHB)
                def _():
                    _issue_xs((b + 1) % RB, par1)

                _drain_rden(b5)
                _drain_xs(b5, par)
                if store_alpha:
                    @pl.when(g >= 2)
                    def _():
                        _drain_alpha(g - 2, par)
                scat = _compute(g, b5, par, last=(b == 9))
            return carry

        lax.fori_loop(0, _JDB, outer, 0)
        if store_alpha:
            _drain_alpha(NCHB - 2, (NCHB - 2) % 2)
            _drain_alpha(NCHB - 1, (NCHB - 1) % 2)
        plsc.subcore_barrier()
        pltpu.sync_copy(out_sh.at[pl.ds(sid * ROWS_T, ROWS_T)],
                        out_part.at[cid, pl.ds(sid * ROWS_T, ROWS_T)])

    return _aggr


_aggr_plain = _make_aggr(False)
_aggr_alpha = _make_aggr(True)


_BN = 1000


def _proj_body(x_ref, w_ref, as_ref, bd_ref, proj_ref, asrc_ref, adst_ref):
    x = x_ref[...]
    t = jnp.dot(x, w_ref[...], preferred_element_type=_f32)
    proj_ref[...] = t
    asrc_ref[...] = jnp.dot(t, as_ref[...], preferred_element_type=_f32)
    adst_ref[...] = jnp.dot(x, bd_ref[...], preferred_element_type=_f32)


def _proj(x, w, amat_src, bmat_dst):
    k = x.shape[1]
    return pl.pallas_call(
        _proj_body,
        out_shape=[
            jax.ShapeDtypeStruct((N, HC), _f32),
            jax.ShapeDtypeStruct((N, LP), _f32),
            jax.ShapeDtypeStruct((N, LP), _f32),
        ],
        grid=(N // _BN,),
        in_specs=[
            pl.BlockSpec((_BN, k), lambda i: (i, 0)),
            pl.BlockSpec((k, HC), lambda i: (0, 0)),
            pl.BlockSpec((HC, LP), lambda i: (0, 0)),
            pl.BlockSpec((k, LP), lambda i: (0, 0)),
        ],
        out_specs=[
            pl.BlockSpec((_BN, HC), lambda i: (i, 0)),
            pl.BlockSpec((_BN, LP), lambda i: (i, 0)),
            pl.BlockSpec((_BN, LP), lambda i: (i, 0)),
        ],
    )(x, w, amat_src, bmat_dst)


def _rden_body(dp_ref, r_ref):
    r_ref[...] = 1.0 / (dp_ref[0] + dp_ref[1] + 1e-16)


def _rden(den_part):
    rb = 1024
    return pl.pallas_call(
        _rden_body,
        out_shape=jax.ShapeDtypeStruct((NP, LP), _f32),
        grid=(NP // rb,),
        in_specs=[pl.BlockSpec((NC, rb, LP), lambda i: (0, i, 0))],
        out_specs=pl.BlockSpec((rb, LP), lambda i: (i, 0)),
    )(den_part)


def _ln(x, g, b):
    mu = jnp.mean(x, axis=-1, keepdims=True)
    var = jnp.mean((x - mu) ** 2, axis=-1, keepdims=True)
    return (x - mu) * lax.rsqrt(var + 1e-5) * g + b


def _epi1_body(op_ref, b_ref, g_ref, be_ref, y_ref):
    o = op_ref[0] + op_ref[1] + b_ref[...]
    o = jnp.where(o >= 0, o, 0.01 * o)
    y_ref[...] = _ln(o, g_ref[...], be_ref[...])


def _epi1(out_part, b, g, be):
    return pl.pallas_call(
        _epi1_body,
        out_shape=jax.ShapeDtypeStruct((N, HC), _f32),
        grid=(N // _BN,),
        in_specs=[
            pl.BlockSpec((NC, _BN, HC), lambda i: (0, i, 0)),
            pl.BlockSpec((HC,), lambda i: (0,)),
            pl.BlockSpec((HC,), lambda i: (0,)),
            pl.BlockSpec((HC,), lambda i: (0,)),
        ],
        out_specs=pl.BlockSpec((_BN, HC), lambda i: (i, 0)),
    )(out_part, b, g, be)


def _epi2_body(op_ref, b_ref, g_ref, be_ref, y_ref):
    o = op_ref[0] + op_ref[1]
    m = o[:, 0:C]
    for h in range(1, H):
        m = m + o[:, h * C:(h + 1) * C]
    m = m * (1.0 / H) + b_ref[...]
    m = jnp.where(m >= 0, m, 0.01 * m)
    y_ref[...] = _ln(m, g_ref[...], be_ref[...])


def _epi2(out_part, b, g, be):
    return pl.pallas_call(
        _epi2_body,
        out_shape=jax.ShapeDtypeStruct((N, C), _f32),
        grid=(N // _BN,),
        in_specs=[
            pl.BlockSpec((NC, _BN, HC), lambda i: (0, i, 0)),
            pl.BlockSpec((C,), lambda i: (0,)),
            pl.BlockSpec((C,), lambda i: (0,)),
            pl.BlockSpec((C,), lambda i: (0,)),
        ],
        out_specs=pl.BlockSpec((_BN, C), lambda i: (i, 0)),
    )(out_part, b, g, be)


def _amat(a):
    e5 = jnp.eye(LP, dtype=_f32)[:H]
    return (a[0][:, :, None] * e5[:, None, :]).reshape(HC, LP)


def _ce_pad(le, ae):
    ce = jnp.sum(le.reshape(H, C) * ae[0], axis=-1)
    return jnp.concatenate([ce, jnp.zeros((LP - H,), _f32)])


def _direction(asrc, adst, s_idx, d_idx, w, cep, proj_src, zeros16, zeros160,
               store_alpha):
    p, den_part = _edge_logits(asrc, adst, s_idx, d_idx, w, cep, zeros16)
    rden = _rden(den_part)
    if store_alpha:
        alpha, out_part = _aggr_alpha(s_idx, d_idx, p, rden, proj_src, zeros160)
        return alpha, out_part
    out_part, = _aggr_plain(s_idx, d_idx, p, rden, proj_src, zeros160)
    return None, out_part


def kernel(x_s, x_t, edge_index, edge_weight, W1, le1, as1, ad1, ae1, b1, g1, be1, W2s, W2d, le2, as2, ad2, ae2, b2, g2, be2):
    src = edge_index[0].astype(jnp.int32)
    dst = edge_index[1].astype(jnp.int32)
    w = edge_weight[:, 0]
    zeros16 = jnp.zeros((NP, LP), _f32)
    zeros160 = jnp.zeros((NP, HC), _f32)

    As1 = _amat(as1)
    Ad1 = _amat(ad1)
    cep1 = _ce_pad(le1, ae1)
    proj_s, asrc_s, adst_s = _proj(x_s, W1, As1, W1 @ Ad1)
    proj_t, asrc_t, adst_t = _proj(x_t, W1, As1, W1 @ Ad1)

    _, op_t = _direction(asrc_s, adst_t, src, dst, w, cep1, proj_s,
                         zeros16, zeros160, False)
    _, op_s = _direction(asrc_t, adst_s, dst, src, w, cep1, proj_t,
                         zeros16, zeros160, False)
    x_t1 = _epi1(op_t, b1, g1, be1)
    x_s1 = _epi1(op_s, b1, g1, be1)

    As2 = _amat(as2)
    Ad2 = _amat(ad2)
    cep2 = _ce_pad(le2, ae2)
    Bd2 = W2d @ Ad2
    proj2_s, asrc2_s, adst2_s = _proj(x_s1, W2s, As2, Bd2)
    proj2_t, asrc2_t, adst2_t = _proj(x_t1, W2s, As2, Bd2)

    al_t, op_t2 = _direction(asrc2_s, adst2_t, src, dst, w, cep2, proj2_s,
                             zeros16, zeros160, True)
    al_s, op_s2 = _direction(asrc2_t, adst2_s, dst, src, w, cep2, proj2_t,
                             zeros16, zeros160, True)
    x_t2 = _epi2(op_t2, b2, g2, be2)
    x_s2 = _epi2(op_s2, b2, g2, be2)
    at_t = al_t[:, :H]
    at_s = al_s[:, :H]
    return (x_s2, at_s, x_t2, at_t)
